# baseline (speedup 1.0000x reference)
import functools

import jax
import jax.numpy as jnp
from jax import lax
from jax.experimental import pallas as pl
from jax.experimental.pallas import tpu as pltpu
from jax.experimental.pallas import tpu_sc as plsc

_NC, _NS = 2, 16
_NW = _NC * _NS


def _sc_gather(xT, i1, i2, i3, i4):
    E, C = xT.shape
    CH = 128
    n_chunks = E // CH
    n_lo = n_chunks // _NW
    extra = n_chunks - n_lo * _NW
    max_ch = n_lo + (1 if extra else 0)

    mesh = plsc.VectorSubcoreMesh(core_axis_name="c", subcore_axis_name="s")

    @functools.partial(
        pl.kernel,
        mesh=mesh,
        compiler_params=pltpu.CompilerParams(use_tc_tiling_on_sc=False),
        out_type=jax.ShapeDtypeStruct((4, E, C), jnp.int32),
        scratch_types=[
            pltpu.VMEM((max_ch * CH,), jnp.int32),
            pltpu.VMEM((max_ch * CH,), jnp.int32),
            pltpu.VMEM((max_ch * CH,), jnp.int32),
            pltpu.VMEM((max_ch * CH,), jnp.int32),
            pltpu.VMEM((4, CH, C), jnp.int32),
            pltpu.SemaphoreType.DMA,
        ],
    )
    def k(xT_hbm, i1_hbm, i2_hbm, i3_hbm, i4_hbm, out_hbm,
          iv1, iv2, iv3, iv4, rows_v, sem):
        wid = lax.axis_index("s") * _NC + lax.axis_index("c")
        start_ch = n_lo * wid + jnp.minimum(wid, extra)
        n_ch = n_lo + jnp.where(wid < extra, 1, 0)
        base = pl.multiple_of(start_ch * CH, CH)
        idx_vs = (iv1, iv2, iv3, iv4)
        for j4, ik in enumerate((i1_hbm, i2_hbm, i3_hbm, i4_hbm)):
            pltpu.sync_copy(ik.at[pl.ds(base, n_lo * CH)],
                            idx_vs[j4].at[pl.ds(0, n_lo * CH)])
        if extra:
            @pl.when(wid < extra)
            def _():
                for j4, ik in enumerate((i1_hbm, i2_hbm, i3_hbm, i4_hbm)):
                    pltpu.sync_copy(
                        ik.at[pl.ds(base + n_lo * CH, CH)],
                        idx_vs[j4].at[pl.ds(n_lo * CH, CH)])

        def chunk(j, carry):
            off = pl.multiple_of(j * CH, CH)
            cps = [
                pltpu.async_copy(
                    xT_hbm.at[idx_vs[j4].at[pl.ds(off, CH)]], rows_v.at[j4],
                    sem,
                )
                for j4 in range(4)
            ]
            for cp in cps:
                cp.wait()
            for j4 in range(4):
                pltpu.sync_copy(
                    rows_v.at[j4], out_hbm.at[j4, pl.ds(base + off, CH)]
                )
            return carry

        lax.fori_loop(0, n_ch, chunk, 0)

    return k(xT, i1, i2, i3, i4)


def _tc_conv(x2d, f4, w0, w14, b2):
    C, E = x2d.shape
    O = w0.shape[0]
    BLK = 640

    def body(x_ref, f_ref, w0_ref, w14_ref, b_ref, o_ref):
        xb = x_ref[...]
        f1 = f_ref[0]
        f2 = f_ref[1]
        f3 = f_ref[2]
        f4_ = f_ref[3]
        s13 = f1 + f3
        s24 = f2 + f4_
        d13 = jnp.abs(f1 - f3)
        d24 = jnp.abs(f2 - f4_)
        w14 = w14_ref[...]

        def mm(kk, feat):
            return lax.dot_general(
                w14[kk], feat, (((1,), (1,)), ((), ())),
                preferred_element_type=jnp.float32,
            )

        acc = lax.dot_general(
            w0_ref[...], xb, (((1,), (0,)), ((), ())),
            preferred_element_type=jnp.float32,
        )
        acc = acc + mm(0, s13) + mm(1, s24) + mm(2, d13) + mm(3, d24)
        o_ref[...] = acc + b_ref[...]

    return pl.pallas_call(
        body,
        grid=(E // BLK,),
        in_specs=[
            pl.BlockSpec((C, BLK), lambda i: (0, i)),
            pl.BlockSpec((4, BLK, C), lambda i: (0, i, 0)),
            pl.BlockSpec((O, C), lambda i: (0, 0)),
            pl.BlockSpec((4, O, C), lambda i: (0, 0, 0)),
            pl.BlockSpec((O, 1), lambda i: (0, 0)),
        ],
        out_specs=pl.BlockSpec((O, BLK), lambda i: (0, i)),
        out_shape=jax.ShapeDtypeStruct((O, E), jnp.float32),
    )(x2d, f4, w0, w14, b2)


def kernel(x, gemm_edges, W, b):
    x2d = x[0, :, :, 0]
    E = x2d.shape[1]
    xT = jnp.transpose(x2d).astype(jnp.bfloat16)
    xp = lax.bitcast_convert_type(
        xT.reshape(E, -1, 2), jnp.int32)
    ge = gemm_edges[0].astype(jnp.int32)
    f4p = _sc_gather(xp, ge[:, 0], ge[:, 1], ge[:, 2], ge[:, 3])
    f4 = lax.bitcast_convert_type(
        f4p, jnp.bfloat16).reshape(4, E, -1)
    w0 = W[:, :, 0, 0]
    w14 = jnp.transpose(W[:, :, 0, 1:], (2, 0, 1)).astype(jnp.bfloat16)
    out = _tc_conv(x2d, f4, w0, w14, b[:, None])
    return out[None, :, :, None]


# speedup vs baseline: 3.0240x; 3.0240x over previous
import functools

import jax
import jax.numpy as jnp
from jax import lax
from jax.experimental import pallas as pl
from jax.experimental.pallas import tpu as pltpu
from jax.experimental.pallas import tpu_sc as plsc

_NC, _NS = 2, 16
_NW = _NC * _NS
_S = 2


def _sc_gather(xT, i1, i2, i3, i4, e0, Es):
    E, C = xT.shape
    CH = 128
    n_chunks = Es // CH
    n_lo = n_chunks // _NW
    extra = n_chunks - n_lo * _NW
    max_ch = n_lo + (1 if extra else 0)

    mesh = plsc.VectorSubcoreMesh(core_axis_name="c", subcore_axis_name="s")

    @functools.partial(
        pl.kernel,
        mesh=mesh,
        out_type=jax.ShapeDtypeStruct((4, Es, C), jnp.float32),
        scratch_types=[
            pltpu.VMEM((max_ch * CH,), jnp.int32),
            pltpu.VMEM((max_ch * CH,), jnp.int32),
            pltpu.VMEM((max_ch * CH,), jnp.int32),
            pltpu.VMEM((max_ch * CH,), jnp.int32),
            pltpu.VMEM((4, CH, C), jnp.float32),
            pltpu.SemaphoreType.DMA,
        ],
    )
    def k(xT_hbm, i1_hbm, i2_hbm, i3_hbm, i4_hbm, out_hbm,
          iv1, iv2, iv3, iv4, rows_v, sem):
        wid = lax.axis_index("s") * _NC + lax.axis_index("c")
        start_ch = n_lo * wid + jnp.minimum(wid, extra)
        n_ch = n_lo + jnp.where(wid < extra, 1, 0)
        base = pl.multiple_of(start_ch * CH, CH)
        idx_vs = (iv1, iv2, iv3, iv4)
        for j4, ik in enumerate((i1_hbm, i2_hbm, i3_hbm, i4_hbm)):
            pltpu.sync_copy(ik.at[pl.ds(e0 + base, n_lo * CH)],
                            idx_vs[j4].at[pl.ds(0, n_lo * CH)])
        if extra:
            @pl.when(wid < extra)
            def _():
                for j4, ik in enumerate((i1_hbm, i2_hbm, i3_hbm, i4_hbm)):
                    pltpu.sync_copy(
                        ik.at[pl.ds(e0 + base + n_lo * CH, CH)],
                        idx_vs[j4].at[pl.ds(n_lo * CH, CH)])

        def chunk(j, carry):
            off = pl.multiple_of(j * CH, CH)
            cps = [
                pltpu.async_copy(
                    xT_hbm.at[idx_vs[j4].at[pl.ds(off, CH)]], rows_v.at[j4],
                    sem,
                )
                for j4 in range(4)
            ]
            for cp in cps:
                cp.wait()
            for j4 in range(4):
                pltpu.sync_copy(
                    rows_v.at[j4], out_hbm.at[j4, pl.ds(base + off, CH)]
                )
            return carry

        lax.fori_loop(0, n_ch, chunk, 0)

    return k(xT, i1, i2, i3, i4)


def _tc_conv(x2d, f4, w0, w14, b2, e0, Es):
    C, E = x2d.shape
    O = w0.shape[0]
    BLK = 640
    blk0 = e0 // BLK

    def body(x_ref, f_ref, w0_ref, w14_ref, b_ref, o_ref):
        xb = x_ref[...]
        f1 = f_ref[0]
        f2 = f_ref[1]
        f3 = f_ref[2]
        f4_ = f_ref[3]
        s13 = f1 + f3
        s24 = f2 + f4_
        d13 = jnp.abs(f1 - f3)
        d24 = jnp.abs(f2 - f4_)
        w14 = w14_ref[...]

        def mm(kk, feat):
            return lax.dot_general(
                w14[kk], feat, (((1,), (1,)), ((), ())),
                preferred_element_type=jnp.float32,
            )

        acc = lax.dot_general(
            w0_ref[...], xb, (((1,), (0,)), ((), ())),
            preferred_element_type=jnp.float32,
        )
        acc = acc + mm(0, s13) + mm(1, s24) + mm(2, d13) + mm(3, d24)
        o_ref[...] = acc + b_ref[...]

    return pl.pallas_call(
        body,
        grid=(Es // BLK,),
        in_specs=[
            pl.BlockSpec((C, BLK), lambda i: (0, i + blk0)),
            pl.BlockSpec((4, BLK, C), lambda i: (0, i, 0)),
            pl.BlockSpec((O, C), lambda i: (0, 0)),
            pl.BlockSpec((4, O, C), lambda i: (0, 0, 0)),
            pl.BlockSpec((O, 1), lambda i: (0, 0)),
        ],
        out_specs=pl.BlockSpec((O, BLK), lambda i: (0, i)),
        out_shape=jax.ShapeDtypeStruct((O, Es), jnp.float32),
    )(x2d, f4, w0, w14, b2)


def kernel(x, gemm_edges, W, b):
    x2d = x[0, :, :, 0]
    E = x2d.shape[1]
    xT = jnp.transpose(x2d)
    ge = gemm_edges[0].astype(jnp.int32)
    i1, i2, i3, i4 = ge[:, 0], ge[:, 1], ge[:, 2], ge[:, 3]
    w0 = W[:, :, 0, 0]
    w14 = jnp.transpose(W[:, :, 0, 1:], (2, 0, 1))
    b2 = b[:, None]
    Es = E // _S
    outs = []
    for s in range(_S):
        f4 = _sc_gather(xT, i1, i2, i3, i4, s * Es, Es)
        outs.append(_tc_conv(x2d, f4, w0, w14, b2, s * Es, Es))
    out = jnp.concatenate(outs, axis=1) if _S > 1 else outs[0]
    return out[None, :, :, None]
